# baseline (speedup 1.0000x reference)
import functools

import jax
import jax.numpy as jnp
from jax import lax
from jax.experimental import pallas as pl
from jax.experimental.pallas import tpu as pltpu
from jax.experimental.pallas import tpu_sc as plsc

_T = 16
_M = 500000
_D = 2
_B = 4096

_NC = 2
_NS = 16
_L = 16
_NOWN = 8
_RANGE = _M // _NOWN
_POSBITS = 12
_G = _B // _L
_HALF = _B // _NC
_GH = _HALF // _L
_NROW = 32
_ROWW = _HALF * _D // _NROW


def _sc_body(out_hbm, idx_hbm, val_hbm,
             lp_sh, idxv, aux, lpown, lpbuf, lpc, valt, stg, eidx, nbuf, sem):
  c = lax.axis_index("c")
  s = lax.axis_index("s")
  iota = lax.iota(jnp.int32, _L)

  pltpu.sync_copy(idx_hbm, idxv)

  @pl.when(s < _NOWN)
  def _owner():
    lo = s * _RANGE

    def g1(g, carry):
      a = idxv[pl.ds(g * _L, _L)]
      pos = g * _L + iota
      key = (a << _POSBITS) | pos
      ks, _ = plsc.sort_key_val(key, key)
      asort = ks >> _POSBITS
      psort = ks & (_B - 1)
      nbuf[...] = asort
      anext = plsc.load_gather(nbuf, [jnp.minimum(iota + 1, _L - 1)])
      rel = asort - lo
      inr = (rel >= 0) & (rel < _RANGE)
      mlast = (asort != anext) | (iota == _L - 1)
      plsc.store_scatter(aux, [jnp.where(inr, rel, 0)], psort,
                         mask=mlast & inr)
      return carry

    lax.fori_loop(0, _G, g1, 0)

    def g2(g, carry):
      a = idxv[pl.ds(g * _L, _L)]
      rel = a - lo
      inr = (rel >= 0) & (rel < _RANGE)
      p = plsc.load_gather(aux, [jnp.where(inr, rel, 0)], mask=inr)
      lpown[pl.ds(g * _L, _L)] = jnp.where(inr, p, 0)
      return carry

    lax.fori_loop(0, _G, g2, 0)
    pltpu.sync_copy(lpown, lp_sh.at[s])

  plsc.subcore_barrier()

  t = s
  o = c * _HALF

  for so in range(_NOWN):
    pltpu.sync_copy(lp_sh.at[so, pl.ds(o, _HALF)], lpbuf.at[so])
  pltpu.sync_copy(val_hbm.at[t], valt)

  def g3(g, carry):
    lp = lpbuf[0, pl.ds(g * _L, _L)]
    for so in range(1, _NOWN):
      lp = lp + lpbuf[so, pl.ds(g * _L, _L)]
    lpc[pl.ds(g * _L, _L)] = lp
    return carry

  lax.fori_loop(0, _GH, g3, 0)

  for g in range(_GH):
    a = idxv[pl.ds(o + g * _L, _L)]
    lp = lpc[pl.ds(g * _L, _L)]
    d0 = plsc.load_gather(valt, [lp * 2])
    d1 = plsc.load_gather(valt, [lp * 2 + 1])
    p0 = (g * _L + iota) * 2
    plsc.store_scatter(stg, [p0], d0)
    plsc.store_scatter(stg, [p0 + 1], d1)
    e2 = (t * _M + a) * 2
    r = g // (_ROWW // (2 * _L))
    rvec = iota * 0 + r
    col0 = (g % (_ROWW // (2 * _L))) * (2 * _L) + iota * 2
    plsc.store_scatter(eidx, [rvec, col0], e2)
    plsc.store_scatter(eidx, [rvec, col0 + 1], e2 + 1)

  copies = []
  for j in range(_NROW):
    copies.append(pltpu.async_copy(
        stg.at[pl.ds(j * _ROWW, _ROWW)], out_hbm.at[eidx.at[j]], sem))
  for cp in copies:
    cp.wait()


@functools.cache
def _build_sc_update():
  return pl.kernel(
    _sc_body,
    out_type=(),
    mesh=plsc.VectorSubcoreMesh(core_axis_name="c", subcore_axis_name="s",
                                num_cores=_NC, num_subcores=_NS),
    compiler_params=pltpu.CompilerParams(needs_layout_passes=False,
                                         use_tc_tiling_on_sc=False),
    scratch_types=[
        pltpu.VMEM_SHARED((_NOWN, _B), jnp.int32),
        pltpu.VMEM((_B,), jnp.int32),
        pltpu.VMEM((_RANGE,), jnp.int32),
        pltpu.VMEM((_B,), jnp.int32),
        pltpu.VMEM((_NOWN, _HALF), jnp.int32),
        pltpu.VMEM((_HALF,), jnp.int32),
        pltpu.VMEM((_B * _D,), jnp.float32),
        pltpu.VMEM((_HALF * _D,), jnp.float32),
        pltpu.VMEM((_NROW, _ROWW), jnp.int32),
        pltpu.VMEM((_L,), jnp.int32),
        pltpu.SemaphoreType.DMA,
    ],
  )


def kernel(mem, idx, val):
  T, M, D = mem.shape
  B = idx.shape[0]
  out_ref = jax.new_ref(mem.reshape(T * M * D))
  _build_sc_update()(out_ref, idx, val.reshape(T, B * D))
  return out_ref[...].reshape(T, M, D)


# speedup vs baseline: 16.2926x; 16.2926x over previous
import functools

import jax
import jax.numpy as jnp
from jax import lax
from jax.experimental import pallas as pl
from jax.experimental.pallas import tpu as pltpu
from jax.experimental.pallas import tpu_sc as plsc

_T = 16
_M = 500000
_D = 2
_B = 4096

_NC = 2
_NS = 16
_L = 16
_RANGE = _M // _NS
_POSBITS = 12
_G = _B // _L
_CHB = 14
_CH = 1 << _CHB
_NCH = (_M + _CH - 1) // _CH
_CPO = (_NCH + _NS - 1) // _NS
_LBLK = 1024
_NBLK = _B // _LBLK
_NBUF = 3


def _sc_body(mem_hbm, idx_hbm, val_hbm, out_hbm,
             lp_sh, cpk_sh, ccnt_sh,
             idxv, aux, lpown, iotav, lpf, cpkv, lstv,
             valrow, buf0, buf1, buf2, cntb, nbuf,
             si0, si1, si2, so0, so1, so2, sema):
  c = lax.axis_index("c")
  s = lax.axis_index("s")
  t = s
  d = c
  iota = lax.iota(jnp.int32, _L)
  bufs = (buf0, buf1, buf2)
  sis = (si0, si1, si2)
  sos = (so0, so1, so2)

  ins = [pltpu.async_copy(mem_hbm.at[t, d, pl.ds(k * _CH, _CH)],
                          bufs[k], sis[k])
         for k in range(_NBUF)]

  pltpu.sync_copy(idx_hbm, idxv)
  pltpu.sync_copy(val_hbm.at[t, d], valrow)

  lo = s * _RANGE

  def g1(g, carry):
    a = idxv[pl.ds(g * _L, _L)]
    pos = g * _L + iota
    key = (a << _POSBITS) | pos
    ks, _ = plsc.sort_key_val(key, key)
    asort = ks >> _POSBITS
    psort = ks & (_B - 1)
    nbuf[...] = asort
    anext = plsc.load_gather(nbuf, [jnp.minimum(iota + 1, _L - 1)])
    rel = asort - lo
    inr = (rel >= 0) & (rel < _RANGE)
    mlast = (asort != anext) | (iota == _L - 1)
    plsc.store_scatter(aux, [jnp.where(inr, rel, 0)], psort, mask=mlast & inr)
    return carry

  lax.fori_loop(0, _G, g1, 0)

  def g2(g, carry):
    a = idxv[pl.ds(g * _L, _L)]
    rel = a - lo
    inr = (rel >= 0) & (rel < _RANGE)
    p = plsc.load_gather(aux, [jnp.where(inr, rel, 0)], mask=inr)
    lpown[pl.ds(g * _L, _L)] = jnp.where(inr, p, 0)
    lpf[pl.ds(g * _L, _L)] = iota * 0
    return carry

  lax.fori_loop(0, _G, g2, 0)

  for g in range(_G):
    iotav[pl.ds(g * _L, _L)] = g * _L + iota

  @pl.when(s == 0)
  def _init():
    pltpu.sync_copy(lpf, lp_sh)

  plsc.subcore_barrier()
  adds = []
  for blk in range(_B // 128):
    adds.append(pltpu.async_copy(
        lpown.at[pl.ds(blk * 128, 128)],
        lp_sh.at[iotav.at[pl.ds(blk * 128, 128)]], sema,
        add=True))
  for a_ in adds:
    a_.wait()
  plsc.subcore_barrier()
  pltpu.sync_copy(lp_sh, lpf)

  for kk in range(_CPO):
    ck = s + kk * _NS

    @pl.when(ck < _NCH)
    def _compact():
      def g3(g, off):
        a = idxv[pl.ds(g * _L, _L)]
        lpv = lpf[pl.ds(g * _L, _L)]
        m = (a >> _CHB) == ck
        mi = jnp.where(m, 1, 0)
        pos = off + plsc.cumsum(mi) - 1
        posc = jnp.where(m, pos, 0)
        plsc.store_scatter(cpkv, [posc], (a << _POSBITS) | lpv, mask=m)
        return off + jnp.sum(mi)

      cnt = lax.fori_loop(0, _G, g3, 0)
      cntb[pl.ds(0, _L)] = iota * 0 + cnt
      pltpu.sync_copy(cntb.at[pl.ds(0, _L)], ccnt_sh.at[pl.ds(ck * _L, _L)])
      nblk = (cnt + _LBLK - 1) // _LBLK
      for b in range(_NBLK):
        @pl.when(b < nblk)
        def _pub():
          pltpu.sync_copy(cpkv.at[pl.ds(b * _LBLK, _LBLK)],
                          cpk_sh.at[pl.ds(ck * _B + b * _LBLK, _LBLK)])

  plsc.subcore_barrier()

  pltpu.sync_copy(ccnt_sh, cntb)

  outs = [None] * _NBUF
  for ck in range(_NCH):
    b = ck % _NBUF
    nxt = ck + _NBUF - 1
    if _NBUF <= nxt < _NCH:
      b2 = nxt % _NBUF
      if outs[b2] is not None:
        outs[b2].wait()
        outs[b2] = None
      m0n = nxt * _CH
      szn = min(_CH, _M - m0n)
      ins[b2] = pltpu.async_copy(
          mem_hbm.at[t, d, pl.ds(m0n, szn)], bufs[b2].at[pl.ds(0, szn)],
          sis[b2])

    m0 = ck * _CH
    sz = min(_CH, _M - m0)
    buf = bufs[b]
    ins[b].wait()

    cnt = cntb[pl.ds(ck * _L, _L)][0]
    nblk = (cnt + _LBLK - 1) // _LBLK
    for bb in range(_NBLK):
      @pl.when(bb < nblk)
      def _pull():
        pltpu.sync_copy(cpk_sh.at[pl.ds(ck * _B + bb * _LBLK, _LBLK)],
                        lstv.at[pl.ds(bb * _LBLK, _LBLK)])

    def upd(g, carry):
      msk = g * _L + iota < cnt
      pk = lstv[pl.ds(g * _L, _L)]
      a = pk >> _POSBITS
      lpv = pk & (_B - 1)
      v = plsc.load_gather(valrow, [jnp.where(msk, lpv, 0)])
      rel = a - m0
      ok = msk & (rel >= 0) & (rel < sz)
      plsc.store_scatter(buf, [jnp.where(ok, rel, 0)], v, mask=ok)
      return carry

    lax.fori_loop(0, (cnt + _L - 1) // _L, upd, 0)

    if outs[b] is not None:
      outs[b].wait()
    outs[b] = pltpu.async_copy(
        buf.at[pl.ds(0, sz)], out_hbm.at[t, d, pl.ds(m0, sz)], sos[b])

  for b in range(_NBUF):
    if outs[b] is not None:
      outs[b].wait()


@functools.cache
def _build_sc_update():
  return pl.kernel(
      _sc_body,
      out_type=jax.ShapeDtypeStruct((_T, _D, _M), jnp.float32),
      mesh=plsc.VectorSubcoreMesh(core_axis_name="c", subcore_axis_name="s",
                                  num_cores=_NC, num_subcores=_NS),
      compiler_params=pltpu.CompilerParams(needs_layout_passes=False,
                                           use_tc_tiling_on_sc=False),
      scratch_types=[
          pltpu.VMEM_SHARED((_B,), jnp.int32),
          pltpu.VMEM_SHARED((_NCH * _B,), jnp.int32),
          pltpu.VMEM_SHARED((_NCH * _L,), jnp.int32),
          pltpu.VMEM((_B,), jnp.int32),
          pltpu.VMEM((_RANGE,), jnp.int32),
          pltpu.VMEM((_B,), jnp.int32),
          pltpu.VMEM((_B,), jnp.int32),
          pltpu.VMEM((_B,), jnp.int32),
          pltpu.VMEM((_B,), jnp.int32),
          pltpu.VMEM((_B,), jnp.int32),
          pltpu.VMEM((_B,), jnp.float32),
          pltpu.VMEM((_CH,), jnp.float32),
          pltpu.VMEM((_CH,), jnp.float32),
          pltpu.VMEM((_CH,), jnp.float32),
          pltpu.VMEM((_NCH * _L,), jnp.int32),
          pltpu.VMEM((_L,), jnp.int32),
          pltpu.SemaphoreType.DMA,
          pltpu.SemaphoreType.DMA,
          pltpu.SemaphoreType.DMA,
          pltpu.SemaphoreType.DMA,
          pltpu.SemaphoreType.DMA,
          pltpu.SemaphoreType.DMA,
          pltpu.SemaphoreType.DMA,
      ],
  )


def kernel(mem, idx, val):
  memT = mem.transpose(0, 2, 1)
  valT = val.transpose(0, 2, 1)
  outT = _build_sc_update()(memT, idx, valT)
  return outT.transpose(0, 2, 1)
